# baseline (speedup 1.0000x reference)
import functools

import jax
import jax.numpy as jnp
from jax import lax
from jax.experimental import pallas as pl
from jax.experimental.pallas import tpu as pltpu
from jax.experimental.pallas import tpu_sc as plsc

_NUM_CORES = 2
_NUM_SUBCORES = 16
_NUM_WORKERS = _NUM_CORES * _NUM_SUBCORES
_SPLIT = 8


@functools.cache
def _make_sc_gather(n_rows, embed_dim):
    rows_per_worker = n_rows // _NUM_WORKERS
    sub_dim = embed_dim // _SPLIT
    subs_per_worker = rows_per_worker * _SPLIT
    mesh = plsc.VectorSubcoreMesh(core_axis_name="c", subcore_axis_name="s")

    @functools.partial(
        pl.kernel,
        mesh=mesh,
        out_type=jax.ShapeDtypeStruct((n_rows * _SPLIT, sub_dim), jnp.float32),
        scratch_types=[
            pltpu.VMEM((subs_per_worker,), jnp.int32),
            pltpu.VMEM((_SPLIT, sub_dim), jnp.float32),
            pltpu.SemaphoreType.DMA,
        ],
    )
    def gather_rows(idx_hbm, table_hbm, out_hbm, idx_v, row_v, sem):
        wid = lax.axis_index("s") * _NUM_CORES + lax.axis_index("c")
        base = wid * subs_per_worker
        pltpu.sync_copy(idx_hbm.at[pl.ds(base, subs_per_worker)], idx_v)

        def body(g, carry):
            pltpu.async_copy(
                table_hbm.at[idx_v.at[pl.ds(g * _SPLIT, _SPLIT)]], row_v, sem
            ).wait()
            pltpu.sync_copy(row_v, out_hbm.at[pl.ds(base + g * _SPLIT, _SPLIT)])
            return carry

        lax.fori_loop(0, rows_per_worker, body, 0)

    return gather_rows


def kernel(prefix, table):
    b, t = prefix.shape
    embed_dim = table.shape[1]
    flat_idx = prefix.reshape(-1).astype(jnp.int32)
    eidx = (flat_idx[:, None] * _SPLIT + jnp.arange(_SPLIT, dtype=jnp.int32)).reshape(-1)
    table_r = table.reshape(table.shape[0] * _SPLIT, embed_dim // _SPLIT)
    out = _make_sc_gather(b * t, embed_dim)(eidx, table_r)
    return out.reshape(b, t, embed_dim)


# speedup vs baseline: 1.0522x; 1.0522x over previous
import functools

import jax
import jax.numpy as jnp
from jax import lax
from jax.experimental import pallas as pl
from jax.experimental.pallas import tpu as pltpu
from jax.experimental.pallas import tpu_sc as plsc

_NUM_CORES = 2
_NUM_SUBCORES = 16
_NUM_WORKERS = _NUM_CORES * _NUM_SUBCORES
_SPLIT = 8


@functools.cache
def _make_sc_gather(n_rows, embed_dim):
    rows_per_worker = n_rows // _NUM_WORKERS
    sub_dim = embed_dim // _SPLIT
    subs_per_worker = rows_per_worker * _SPLIT
    mesh = plsc.VectorSubcoreMesh(core_axis_name="c", subcore_axis_name="s")

    @functools.partial(
        pl.kernel,
        mesh=mesh,
        out_type=jax.ShapeDtypeStruct((n_rows * _SPLIT, sub_dim), jnp.float32),
        scratch_types=[
            pltpu.VMEM((subs_per_worker,), jnp.int32),
            pltpu.VMEM((2, _SPLIT, sub_dim), jnp.float32),
            pltpu.SemaphoreType.DMA,
            pltpu.SemaphoreType.DMA,
        ],
    )
    def gather_rows(idx_hbm, table_hbm, out_hbm, idx_v, rows_v, sem0, sem1):
        wid = lax.axis_index("s") * _NUM_CORES + lax.axis_index("c")
        base = wid * subs_per_worker
        sems = (sem0, sem1)
        pltpu.sync_copy(idx_hbm.at[pl.ds(base, subs_per_worker)], idx_v)

        def start_gather(g, b):
            pltpu.async_copy(
                table_hbm.at[idx_v.at[pl.ds(g * _SPLIT, _SPLIT)]],
                rows_v.at[b],
                sems[b],
            )

        def wait_gather(b):
            pltpu.make_async_copy(
                table_hbm.at[pl.ds(0, _SPLIT)], rows_v.at[b], sems[b]
            ).wait()

        start_gather(0, 0)
        start_gather(1, 1)
        n_pairs = rows_per_worker // 2

        def body(k, carry):
            for b in range(2):
                g = 2 * k + b
                wait_gather(b)
                pltpu.sync_copy(
                    rows_v.at[b], out_hbm.at[pl.ds(base + g * _SPLIT, _SPLIT)]
                )

                @pl.when(k < n_pairs - 1)
                def _():
                    start_gather(g + 2, b)

            return carry

        lax.fori_loop(0, n_pairs, body, 0)

    return gather_rows


def kernel(prefix, table):
    b, t = prefix.shape
    embed_dim = table.shape[1]
    flat_idx = prefix.reshape(-1).astype(jnp.int32)
    eidx = (flat_idx[:, None] * _SPLIT + jnp.arange(_SPLIT, dtype=jnp.int32)).reshape(-1)
    table_r = table.reshape(table.shape[0] * _SPLIT, embed_dim // _SPLIT)
    out = _make_sc_gather(b * t, embed_dim)(eidx, table_r)
    return out.reshape(b, t, embed_dim)


# speedup vs baseline: 2.3756x; 2.2578x over previous
import functools

import jax
import jax.numpy as jnp
from jax import lax
from jax.experimental import pallas as pl
from jax.experimental.pallas import tpu as pltpu
from jax.experimental.pallas import tpu_sc as plsc

_NUM_CORES = 2
_NUM_SUBCORES = 16
_NUM_WORKERS = _NUM_CORES * _NUM_SUBCORES


@functools.cache
def _make_sc_gather(n_rows, embed_dim):
    rows_per_worker = n_rows // _NUM_WORKERS
    mesh = plsc.VectorSubcoreMesh(core_axis_name="c", subcore_axis_name="s")

    @functools.partial(
        pl.kernel,
        mesh=mesh,
        out_type=jax.ShapeDtypeStruct((n_rows, embed_dim), jnp.float32),
        scratch_types=[
            pltpu.VMEM((rows_per_worker,), jnp.int32),
            pltpu.VMEM((2, 1, embed_dim), jnp.float32),
            pltpu.SemaphoreType.DMA,
            pltpu.SemaphoreType.DMA,
        ],
    )
    def gather_rows(idx_hbm, table_hbm, out_hbm, idx_v, rows_v, sem0, sem1):
        wid = lax.axis_index("s") * _NUM_CORES + lax.axis_index("c")
        base = wid * rows_per_worker
        sems = (sem0, sem1)
        pltpu.sync_copy(idx_hbm.at[pl.ds(base, rows_per_worker)], idx_v)

        def start_gather(row, b):
            pltpu.async_copy(
                table_hbm.at[pl.ds(row, 1)], rows_v.at[b], sems[b]
            )

        def wait_gather(b):
            pltpu.make_async_copy(
                table_hbm.at[pl.ds(0, 1)], rows_v.at[b], sems[b]
            ).wait()

        n_chunks = rows_per_worker // 16

        def chunk_body(c, carry):
            vec = idx_v[pl.ds(c * 16, 16)]
            start_gather(vec[0], 0)
            start_gather(vec[1], 1)
            for j in range(16):
                b = j % 2
                wait_gather(b)
                pltpu.sync_copy(
                    rows_v.at[b], out_hbm.at[pl.ds(base + c * 16 + j, 1)]
                )
                if j + 2 < 16:
                    start_gather(vec[j + 2], b)
            return carry

        lax.fori_loop(0, n_chunks, chunk_body, 0)

    return gather_rows


def kernel(prefix, table):
    b, t = prefix.shape
    embed_dim = table.shape[1]
    flat_idx = prefix.reshape(-1).astype(jnp.int32)
    out = _make_sc_gather(b * t, embed_dim)(flat_idx, table)
    return out.reshape(b, t, embed_dim)


# speedup vs baseline: 3.4862x; 1.4675x over previous
import functools

import jax
import jax.numpy as jnp
from jax import lax
from jax.experimental import pallas as pl
from jax.experimental.pallas import tpu as pltpu
from jax.experimental.pallas import tpu_sc as plsc

_NUM_CORES = 2
_NUM_SUBCORES = 16
_NUM_WORKERS = _NUM_CORES * _NUM_SUBCORES


@functools.cache
def _make_sc_gather(n_rows, embed_dim):
    rows_per_worker = n_rows // _NUM_WORKERS
    n_chunks = rows_per_worker // 16
    mesh = plsc.VectorSubcoreMesh(core_axis_name="c", subcore_axis_name="s")

    @functools.partial(
        pl.kernel,
        mesh=mesh,
        out_type=jax.ShapeDtypeStruct((n_rows, embed_dim), jnp.float32),
        scratch_types=[
            pltpu.VMEM((rows_per_worker,), jnp.int32),
            pltpu.VMEM((rows_per_worker,), jnp.int32),
            pltpu.VMEM((1, embed_dim), jnp.float32),
        ],
    )
    def gather_rows(sidx_hbm, dst_hbm, table_hbm, out_hbm, sidx_v, dst_v, buf):
        wid = lax.axis_index("s") * _NUM_CORES + lax.axis_index("c")
        base = wid * rows_per_worker
        pltpu.sync_copy(sidx_hbm.at[pl.ds(base, rows_per_worker)], sidx_v)
        pltpu.sync_copy(dst_hbm.at[pl.ds(base, rows_per_worker)], dst_v)

        def chunk_body(c, prev):
            svec = sidx_v[pl.ds(c * 16, 16)]
            dvec = dst_v[pl.ds(c * 16, 16)]
            for j in range(16):
                row = svec[j]
                prev_row = prev if j == 0 else svec[j - 1]

                @pl.when(row != prev_row)
                def _():
                    pltpu.sync_copy(table_hbm.at[pl.ds(row, 1)], buf)

                pltpu.sync_copy(buf, out_hbm.at[pl.ds(dvec[j], 1)])
            return svec[15]

        lax.fori_loop(0, n_chunks, chunk_body, jnp.int32(-1))

    return gather_rows


def kernel(prefix, table):
    b, t = prefix.shape
    embed_dim = table.shape[1]
    flat_idx = prefix.reshape(-1).astype(jnp.int32)
    order = jnp.argsort(flat_idx).astype(jnp.int32)
    sidx = jnp.take(flat_idx, order)
    out = _make_sc_gather(b * t, embed_dim)(sidx, order, table)
    return out.reshape(b, t, embed_dim)


# speedup vs baseline: 3.7160x; 1.0659x over previous
import functools

import jax
import jax.numpy as jnp
from jax import lax
from jax.experimental import pallas as pl
from jax.experimental.pallas import tpu as pltpu
from jax.experimental.pallas import tpu_sc as plsc

_NUM_CORES = 2
_NUM_SUBCORES = 16
_NUM_WORKERS = _NUM_CORES * _NUM_SUBCORES


@functools.cache
def _make_sc_gather(n_rows, embed_dim):
    rows_per_worker = n_rows // _NUM_WORKERS
    n_chunks = rows_per_worker // 16
    mesh = plsc.VectorSubcoreMesh(core_axis_name="c", subcore_axis_name="s")

    @functools.partial(
        pl.kernel,
        mesh=mesh,
        out_type=jax.ShapeDtypeStruct((n_rows, embed_dim), jnp.float32),
        scratch_types=[
            pltpu.VMEM((rows_per_worker,), jnp.int32),
            pltpu.VMEM((rows_per_worker,), jnp.int32),
            pltpu.VMEM((1, embed_dim), jnp.float32),
        ],
    )
    def gather_rows(sidx_hbm, dst_hbm, table_hbm, out_hbm, sidx_v, dst_v, buf):
        wid = lax.axis_index("s") * _NUM_CORES + lax.axis_index("c")
        base = wid * rows_per_worker
        pltpu.sync_copy(sidx_hbm.at[pl.ds(base, rows_per_worker)], sidx_v)
        pltpu.sync_copy(dst_hbm.at[pl.ds(base, rows_per_worker)], dst_v)

        def chunk_body(c, prev):
            svec = sidx_v[pl.ds(c * 16, 16)]
            dvec = dst_v[pl.ds(c * 16, 16)]
            for j in range(16):
                row = svec[j]
                prev_row = prev if j == 0 else svec[j - 1]

                @pl.when(row != prev_row)
                def _():
                    pltpu.sync_copy(table_hbm.at[pl.ds(row, 1)], buf)

                pltpu.sync_copy(buf, out_hbm.at[pl.ds(dvec[j], 1)])
            return svec[15]

        lax.fori_loop(0, n_chunks, chunk_body, jnp.int32(-1))

    return gather_rows


def kernel(prefix, table):
    b, t = prefix.shape
    embed_dim = table.shape[1]
    flat_idx = prefix.reshape(-1).astype(jnp.int32)
    sidx, order = lax.sort_key_val(
        flat_idx, jnp.arange(flat_idx.shape[0], dtype=jnp.int32)
    )
    out = _make_sc_gather(b * t, embed_dim)(sidx, order, table)
    return out.reshape(b, t, embed_dim)
